# baseline (speedup 1.0000x reference)
import jax
import jax.numpy as jnp
from jax.experimental import pallas as pl
from jax.experimental.pallas import tpu as pltpu

_BN = 4000
_NBUF = 4


def _stream_kernel(p_hbm, q_hbm, ic_hbm, w_ref, weu_ref, wei_ref,
                   uo_hbm, io_hbm,
                   p_buf, q_buf, ic_buf, uo_buf, io_buf,
                   in_sems, out_sems):
    n = p_hbm.shape[0]
    nchunk = n // _BN
    f32 = jnp.float32
    w_fold = jnp.dot(w_ref[...], wei_ref[64:128, :], preferred_element_type=f32)
    wei_top = wei_ref[0:64, :]
    weu = weu_ref[...]

    def in_copies(slot, i):
        r = pl.ds(i * _BN, _BN)
        return (
            pltpu.make_async_copy(p_hbm.at[r, :], p_buf.at[slot], in_sems.at[slot, 0]),
            pltpu.make_async_copy(q_hbm.at[r, :], q_buf.at[slot], in_sems.at[slot, 1]),
            pltpu.make_async_copy(ic_hbm.at[r, :], ic_buf.at[slot], in_sems.at[slot, 2]),
        )

    def out_copies(slot, i):
        r = pl.ds(i * _BN, _BN)
        return (
            pltpu.make_async_copy(uo_buf.at[slot], uo_hbm.at[r, :], out_sems.at[slot, 0]),
            pltpu.make_async_copy(io_buf.at[slot], io_hbm.at[r, :], out_sems.at[slot, 1]),
        )

    for k in range(min(_NBUF, nchunk)):
        for c in in_copies(k, k):
            c.start()

    for i in range(nchunk):
        slot = i % _NBUF
        for c in in_copies(slot, i):
            c.wait()
        if i >= _NBUF:
            for c in out_copies(slot, i - _NBUF):
                c.wait()
        uo_buf[slot] = jnp.dot(p_buf[slot], weu, preferred_element_type=f32)
        io_buf[slot] = (
            jnp.dot(q_buf[slot], wei_top, preferred_element_type=f32)
            + jnp.dot(ic_buf[slot], w_fold, preferred_element_type=f32)
        )
        for c in out_copies(slot, i):
            c.start()
        nxt = i + _NBUF
        if nxt < nchunk:
            for c in in_copies(slot, nxt):
                c.start()

    for i in range(max(nchunk - _NBUF, 0), nchunk):
        for c in out_copies(i % _NBUF, i):
            c.wait()


@jax.jit
def kernel(P, Q, item_content, W, weu, wei):
    n = P.shape[0]
    d = weu.shape[1]
    f32 = jnp.float32
    any_spec = pl.BlockSpec(memory_space=pl.ANY)
    vmem_spec = pl.BlockSpec(memory_space=pltpu.MemorySpace.VMEM)
    user_emb, item_emb = pl.pallas_call(
        _stream_kernel,
        in_specs=[any_spec, any_spec, any_spec, vmem_spec, vmem_spec, vmem_spec],
        out_specs=[any_spec, any_spec],
        out_shape=[
            jax.ShapeDtypeStruct((n, d), f32),
            jax.ShapeDtypeStruct((n, d), f32),
        ],
        scratch_shapes=[
            pltpu.VMEM((_NBUF, _BN, P.shape[1]), f32),
            pltpu.VMEM((_NBUF, _BN, Q.shape[1]), f32),
            pltpu.VMEM((_NBUF, _BN, item_content.shape[1]), f32),
            pltpu.VMEM((_NBUF, _BN, d), f32),
            pltpu.VMEM((_NBUF, _BN, d), f32),
            pltpu.SemaphoreType.DMA((_NBUF, 3)),
            pltpu.SemaphoreType.DMA((_NBUF, 2)),
        ],
    )(P, Q, item_content, W, weu, wei)
    return (user_emb, item_emb)
